# Initial kernel scaffold; baseline (speedup 1.0000x reference)
import functools

import jax
import jax.numpy as jnp
from jax import lax
from jax.experimental import pallas as pl
from jax.experimental.pallas import tpu as pltpu
from jax.experimental.pallas import tpu_sc as plsc

N = 10000
E = 320000
D_IN = 128
D_HID = 256
D_OUT = 128

W = 144
NC, NS = 2, 16
CHUNK = 128
N_ACC = 10016
ROWS_PER_TILE = N_ACC // NS
E_PAD = 323584
EDGES_PER_TILE = E_PAD // (NC * NS)
N_CHUNKS = EDGES_PER_TILE // CHUNK

_HIGH = lax.Precision.HIGHEST


def _sc_mesh():
    return plsc.VectorSubcoreMesh(core_axis_name="c", subcore_axis_name="s")


def _sage_sc(x_pad, src, dst, zeros):
    @functools.partial(
        pl.kernel,
        out_type=jax.ShapeDtypeStruct((NC, N_ACC, W), jnp.float32),
        mesh=_sc_mesh(),
        scratch_types=[
            pltpu.VMEM_SHARED((N_ACC, W), jnp.float32),
            pltpu.VMEM((CHUNK,), jnp.int32),
            pltpu.VMEM((CHUNK,), jnp.int32),
            pltpu.VMEM((CHUNK, W), jnp.float32),
            pltpu.SemaphoreType.DMA,
        ],
    )
    def k(x_hbm, src_hbm, dst_hbm, zero_hbm, out_hbm, acc, src_v, dst_v, rows_v, sem):
        c = lax.axis_index("c")
        s = lax.axis_index("s")
        row0 = s * ROWS_PER_TILE
        pltpu.sync_copy(zero_hbm.at[pl.ds(row0, ROWS_PER_TILE)],
                        acc.at[pl.ds(row0, ROWS_PER_TILE)])
        plsc.subcore_barrier()
        tile_base = c * (E_PAD // NC) + s * EDGES_PER_TILE

        @pl.loop(0, N_CHUNKS)
        def _(kk):
            base = tile_base + kk * CHUNK
            pltpu.sync_copy(src_hbm.at[pl.ds(base, CHUNK)], src_v)
            pltpu.sync_copy(dst_hbm.at[pl.ds(base, CHUNK)], dst_v)
            pltpu.async_copy(x_hbm.at[src_v], rows_v, sem).wait()
            pltpu.sync_copy(rows_v, acc.at[dst_v], add=True)

        plsc.subcore_barrier()
        pltpu.sync_copy(acc.at[pl.ds(row0, ROWS_PER_TILE)],
                        out_hbm.at[c].at[pl.ds(row0, ROWS_PER_TILE)])

    return k(x_pad, src, dst, zeros)


def _gat_sc(hw_pad, src, dst, a_src, a_dst, zeros):
    @functools.partial(
        pl.kernel,
        out_type=jax.ShapeDtypeStruct((NC, N_ACC, W), jnp.float32),
        mesh=_sc_mesh(),
        scratch_types=[
            pltpu.VMEM_SHARED((N_ACC, W), jnp.float32),
            pltpu.VMEM((CHUNK,), jnp.int32),
            pltpu.VMEM((CHUNK,), jnp.int32),
            pltpu.VMEM((CHUNK, W), jnp.float32),
            pltpu.VMEM((CHUNK,), jnp.float32),
            pltpu.VMEM((N_ACC,), jnp.float32),
            pltpu.VMEM((N_ACC,), jnp.float32),
            pltpu.SemaphoreType.DMA,
        ],
    )
    def k(hw_hbm, src_hbm, dst_hbm, asrc_hbm, adst_hbm, zero_hbm, out_hbm,
          acc, src_v, dst_v, rows_v, ex_v, asrc_v, adst_v, sem):
        c = lax.axis_index("c")
        s = lax.axis_index("s")
        row0 = s * ROWS_PER_TILE
        pltpu.sync_copy(zero_hbm.at[pl.ds(row0, ROWS_PER_TILE)],
                        acc.at[pl.ds(row0, ROWS_PER_TILE)])
        pltpu.sync_copy(asrc_hbm, asrc_v)
        pltpu.sync_copy(adst_hbm, adst_v)
        plsc.subcore_barrier()
        tile_base = c * (E_PAD // NC) + s * EDGES_PER_TILE

        @pl.loop(0, N_CHUNKS)
        def _(kk):
            base = tile_base + kk * CHUNK
            pltpu.sync_copy(src_hbm.at[pl.ds(base, CHUNK)], src_v)
            pltpu.sync_copy(dst_hbm.at[pl.ds(base, CHUNK)], dst_v)
            gat = pltpu.async_copy(hw_hbm.at[src_v], rows_v, sem)
            for g in range(CHUNK // 16):
                si = src_v[pl.ds(g * 16, 16)]
                di = dst_v[pl.ds(g * 16, 16)]
                av = plsc.load_gather(asrc_v, [si])
                bv = plsc.load_gather(adst_v, [di])
                e = av + bv
                e = jnp.maximum(e, e * 0.2)
                ex_v[pl.ds(g * 16, 16)] = jnp.exp(e)
            gat.wait()

            @pl.loop(0, CHUNK)
            def _(i):
                spl = plsc.load_gather(ex_v, [jnp.full((16,), i, jnp.int32)])
                for cg in range(W // 16):
                    rows_v[i, pl.ds(cg * 16, 16)] = (
                        rows_v[i, pl.ds(cg * 16, 16)] * spl)

            pltpu.sync_copy(rows_v, acc.at[dst_v], add=True)

        plsc.subcore_barrier()
        pltpu.sync_copy(acc.at[pl.ds(row0, ROWS_PER_TILE)],
                        out_hbm.at[c].at[pl.ds(row0, ROWS_PER_TILE)])

    return k(hw_pad, src, dst, a_src, a_dst, zeros)


BR = 1000


def _tc_a(p0, p1, x, w_l, w_r, b_s, w_g, a_s, a_d, w_lin, b_lin):
    def body(p0_r, p1_r, x_r, wl_r, wr_r, bs_r, wg_r, as_r, ad_r, wlin_r,
             blin_r, hw_r, skip_r, aux_r):
        s = p0_r[...] + p1_r[...]
        agg = s[:, :D_IN]
        cnt = s[:, D_IN:D_IN + 1]
        mean = agg / jnp.maximum(cnt, 1.0)
        h = (jnp.dot(mean, wl_r[...], precision=_HIGH)
             + jnp.dot(x_r[...], wr_r[...], precision=_HIGH) + bs_r[...])
        h = jnp.maximum(h, 0.0)
        hw = jnp.dot(h, wg_r[...], precision=_HIGH)
        skip_r[...] = jnp.dot(h, wlin_r[...], precision=_HIGH) + blin_r[...]
        av = jnp.sum(hw * as_r[...], axis=1, keepdims=True)
        dv = jnp.sum(hw * ad_r[...], axis=1, keepdims=True)
        hw_r[...] = jnp.concatenate(
            [hw, jnp.ones((BR, W - D_OUT), jnp.float32)], axis=1)
        aux_r[...] = jnp.concatenate(
            [av, dv, jnp.zeros((BR, 126), jnp.float32)], axis=1)

    full = lambda shp: pl.BlockSpec(shp, lambda i: (0,) * len(shp))
    return pl.pallas_call(
        body,
        grid=(N // BR,),
        in_specs=[
            pl.BlockSpec((BR, W), lambda i: (i, 0)),
            pl.BlockSpec((BR, W), lambda i: (i, 0)),
            pl.BlockSpec((BR, D_IN), lambda i: (i, 0)),
            full((D_IN, D_HID)),
            full((D_IN, D_HID)),
            full((1, D_HID)),
            full((D_HID, D_OUT)),
            full((1, D_OUT)),
            full((1, D_OUT)),
            full((D_HID, D_OUT)),
            full((1, D_OUT)),
        ],
        out_specs=[
            pl.BlockSpec((BR, W), lambda i: (i, 0)),
            pl.BlockSpec((BR, D_OUT), lambda i: (i, 0)),
            pl.BlockSpec((BR, 128), lambda i: (i, 0)),
        ],
        out_shape=[
            jax.ShapeDtypeStruct((N, W), jnp.float32),
            jax.ShapeDtypeStruct((N, D_OUT), jnp.float32),
            jax.ShapeDtypeStruct((N, 128), jnp.float32),
        ],
    )(p0, p1, x, w_l, w_r, b_s, w_g, a_s, a_d, w_lin, b_lin)


def _tc_b(q0, q1, skip, b_g):
    def body(q0_r, q1_r, skip_r, bg_r, out_r):
        s = q0_r[...] + q1_r[...]
        denom = jnp.maximum(s[:, D_OUT:D_OUT + 1], 1e-16)
        out_r[...] = s[:, :D_OUT] / denom + bg_r[...] + skip_r[...]

    return pl.pallas_call(
        body,
        grid=(N // BR,),
        in_specs=[
            pl.BlockSpec((BR, W), lambda i: (i, 0)),
            pl.BlockSpec((BR, W), lambda i: (i, 0)),
            pl.BlockSpec((BR, D_OUT), lambda i: (i, 0)),
            pl.BlockSpec((1, D_OUT), lambda i: (0, 0)),
        ],
        out_specs=pl.BlockSpec((BR, D_OUT), lambda i: (i, 0)),
        out_shape=jax.ShapeDtypeStruct((N, D_OUT), jnp.float32),
    )(q0, q1, skip, b_g)


def kernel(x, edge_index, W_sage_l, W_sage_r, b_sage, W_gat, att_src,
           att_dst, b_gat, W_lin, b_lin):
    src = edge_index[0].astype(jnp.int32)
    dst = edge_index[1].astype(jnp.int32)
    pad = E_PAD - E
    src_p = jnp.concatenate([src, jnp.zeros((pad,), jnp.int32)])
    dst_p = jnp.concatenate([dst, jnp.full((pad,), N, jnp.int32)])
    x_pad = jnp.concatenate([x, jnp.ones((N, W - D_IN), jnp.float32)], axis=1)
    zeros = jnp.zeros((N_ACC, W), jnp.float32)

    p = _sage_sc(x_pad, src_p, dst_p, zeros)
    hw_pad, skip, aux = _tc_a(
        p[0, :N], p[1, :N], x, W_sage_l, W_sage_r, b_sage.reshape(1, -1),
        W_gat, att_src.reshape(1, -1), att_dst.reshape(1, -1), W_lin,
        b_lin.reshape(1, -1))
    a_s = jnp.pad(aux[:, 0], (0, N_ACC - N))
    a_d = jnp.pad(aux[:, 1], (0, N_ACC - N))
    q = _gat_sc(hw_pad, src_p, dst_p, a_s, a_d, zeros)
    out = _tc_b(q[0, :N], q[1, :N], skip, b_gat.reshape(1, -1))
    return out


# speedup vs baseline: 10.2528x; 10.2528x over previous
import functools

import jax
import jax.numpy as jnp
from jax import lax
from jax.experimental import pallas as pl
from jax.experimental.pallas import tpu as pltpu
from jax.experimental.pallas import tpu_sc as plsc

N = 10000
E = 320000
D_IN = 128
D_HID = 256
D_OUT = 128

W = 144
NC, NS = 2, 16
CHUNK = 128
N_ACC = 10112
ROWS_PER_TILE = N_ACC // NS
E_PAD = 323584
EDGES_PER_TILE = E_PAD // (NC * NS)
N_CHUNKS = EDGES_PER_TILE // CHUNK

_HIGH = lax.Precision.HIGHEST


def _sc_mesh():
    return plsc.VectorSubcoreMesh(core_axis_name="c", subcore_axis_name="s")


def _sage_sc(x_pad, src, dst, zeros):
    @functools.partial(
        pl.kernel,
        out_type=jax.ShapeDtypeStruct((NC, N_ACC, W), jnp.float32),
        mesh=_sc_mesh(),
        compiler_params=pltpu.CompilerParams(use_tc_tiling_on_sc=False),
        scratch_types=[
            pltpu.VMEM_SHARED((N_ACC, W), jnp.float32),
            pltpu.VMEM((CHUNK,), jnp.int32),
            pltpu.VMEM((CHUNK,), jnp.int32),
            pltpu.VMEM((CHUNK, W), jnp.float32),
            pltpu.SemaphoreType.DMA,
        ],
    )
    def k(x_hbm, src_hbm, dst_hbm, zero_hbm, out_hbm, acc, src_v, dst_v, rows_v, sem):
        c = lax.axis_index("c")
        s = lax.axis_index("s")
        row0 = s * ROWS_PER_TILE
        pltpu.sync_copy(zero_hbm.at[pl.ds(row0, ROWS_PER_TILE)],
                        acc.at[pl.ds(row0, ROWS_PER_TILE)])
        plsc.subcore_barrier()
        tile_base = c * (E_PAD // NC) + s * EDGES_PER_TILE

        @pl.loop(0, N_CHUNKS)
        def _(kk):
            base = tile_base + kk * CHUNK
            pltpu.sync_copy(src_hbm.at[pl.ds(base, CHUNK)], src_v)
            pltpu.sync_copy(dst_hbm.at[pl.ds(base, CHUNK)], dst_v)
            pltpu.async_copy(x_hbm.at[src_v], rows_v, sem).wait()
            pltpu.sync_copy(rows_v, acc.at[dst_v], add=True)

        plsc.subcore_barrier()
        pltpu.sync_copy(acc.at[pl.ds(row0, ROWS_PER_TILE)],
                        out_hbm.at[c].at[pl.ds(row0, ROWS_PER_TILE)])

    return k(x_pad, src, dst, zeros)


def _gat_sc(hw_pad, src, dst, a_src, a_dst, zeros):
    @functools.partial(
        pl.kernel,
        out_type=jax.ShapeDtypeStruct((NC, N_ACC, W), jnp.float32),
        mesh=_sc_mesh(),
        compiler_params=pltpu.CompilerParams(
            use_tc_tiling_on_sc=False, needs_layout_passes=False),
        scratch_types=[
            pltpu.VMEM_SHARED((N_ACC, W), jnp.float32),
            pltpu.VMEM((CHUNK,), jnp.int32),
            pltpu.VMEM((CHUNK,), jnp.int32),
            pltpu.VMEM((CHUNK, W), jnp.float32),
            pltpu.VMEM((CHUNK,), jnp.float32),
            pltpu.VMEM((N_ACC,), jnp.float32),
            pltpu.VMEM((N_ACC,), jnp.float32),
            pltpu.SemaphoreType.DMA,
        ],
    )
    def k(hw_hbm, src_hbm, dst_hbm, asrc_hbm, adst_hbm, zero_hbm, out_hbm,
          acc, src_v, dst_v, rows_v, ex_v, asrc_v, adst_v, sem):
        c = lax.axis_index("c")
        s = lax.axis_index("s")
        row0 = s * ROWS_PER_TILE
        pltpu.sync_copy(zero_hbm.at[pl.ds(row0, ROWS_PER_TILE)],
                        acc.at[pl.ds(row0, ROWS_PER_TILE)])
        pltpu.sync_copy(asrc_hbm, asrc_v)
        pltpu.sync_copy(adst_hbm, adst_v)
        plsc.subcore_barrier()
        tile_base = c * (E_PAD // NC) + s * EDGES_PER_TILE

        @pl.loop(0, N_CHUNKS)
        def _(kk):
            base = tile_base + kk * CHUNK
            pltpu.sync_copy(src_hbm.at[pl.ds(base, CHUNK)], src_v)
            pltpu.sync_copy(dst_hbm.at[pl.ds(base, CHUNK)], dst_v)
            gat = pltpu.async_copy(hw_hbm.at[src_v], rows_v, sem)
            for g in range(CHUNK // 16):
                si = src_v[pl.ds(g * 16, 16)]
                di = dst_v[pl.ds(g * 16, 16)]
                av = plsc.load_gather(asrc_v, [si])
                bv = plsc.load_gather(adst_v, [di])
                e = av + bv
                e = jnp.maximum(e, e * 0.2)
                ex_v[pl.ds(g * 16, 16)] = jnp.exp(e)
            gat.wait()

            @pl.loop(0, CHUNK)
            def _(i):
                spl = plsc.load_gather(ex_v, [jnp.full((16,), i, jnp.int32)])
                for cg in range(W // 16):
                    rows_v[i, pl.ds(cg * 16, 16)] = (
                        rows_v[i, pl.ds(cg * 16, 16)] * spl)

            pltpu.sync_copy(rows_v, acc.at[dst_v], add=True)

        plsc.subcore_barrier()
        pltpu.sync_copy(acc.at[pl.ds(row0, ROWS_PER_TILE)],
                        out_hbm.at[c].at[pl.ds(row0, ROWS_PER_TILE)])

    return k(hw_pad, src, dst, a_src, a_dst, zeros)


BR = 1000


def _tc_a(p0, p1, x, w_l, w_r, b_s, w_g, a_s, a_d, w_lin, b_lin):
    def body(p0_r, p1_r, x_r, wl_r, wr_r, bs_r, wg_r, as_r, ad_r, wlin_r,
             blin_r, hw_r, skip_r, aux_r):
        s = p0_r[...] + p1_r[...]
        agg = s[:, :D_IN]
        cnt = s[:, D_IN:D_IN + 1]
        mean = agg / jnp.maximum(cnt, 1.0)
        h = (jnp.dot(mean, wl_r[...], precision=_HIGH)
             + jnp.dot(x_r[...], wr_r[...], precision=_HIGH) + bs_r[...])
        h = jnp.maximum(h, 0.0)
        hw = jnp.dot(h, wg_r[...], precision=_HIGH)
        skip_r[...] = jnp.dot(h, wlin_r[...], precision=_HIGH) + blin_r[...]
        av = jnp.sum(hw * as_r[...], axis=1, keepdims=True)
        dv = jnp.sum(hw * ad_r[...], axis=1, keepdims=True)
        hw_r[...] = jnp.concatenate(
            [hw, jnp.ones((BR, W - D_OUT), jnp.float32)], axis=1)
        aux_r[...] = jnp.concatenate(
            [av, dv, jnp.zeros((BR, 126), jnp.float32)], axis=1)

    full = lambda shp: pl.BlockSpec(shp, lambda i: (0,) * len(shp))
    return pl.pallas_call(
        body,
        grid=(N // BR,),
        in_specs=[
            pl.BlockSpec((BR, W), lambda i: (i, 0)),
            pl.BlockSpec((BR, W), lambda i: (i, 0)),
            pl.BlockSpec((BR, D_IN), lambda i: (i, 0)),
            full((D_IN, D_HID)),
            full((D_IN, D_HID)),
            full((1, D_HID)),
            full((D_HID, D_OUT)),
            full((1, D_OUT)),
            full((1, D_OUT)),
            full((D_HID, D_OUT)),
            full((1, D_OUT)),
        ],
        out_specs=[
            pl.BlockSpec((BR, W), lambda i: (i, 0)),
            pl.BlockSpec((BR, D_OUT), lambda i: (i, 0)),
            pl.BlockSpec((BR, 128), lambda i: (i, 0)),
        ],
        out_shape=[
            jax.ShapeDtypeStruct((N, W), jnp.float32),
            jax.ShapeDtypeStruct((N, D_OUT), jnp.float32),
            jax.ShapeDtypeStruct((N, 128), jnp.float32),
        ],
    )(p0, p1, x, w_l, w_r, b_s, w_g, a_s, a_d, w_lin, b_lin)


def _tc_b(q0, q1, skip, b_g):
    def body(q0_r, q1_r, skip_r, bg_r, out_r):
        s = q0_r[...] + q1_r[...]
        denom = jnp.maximum(s[:, D_OUT:D_OUT + 1], 1e-16)
        out_r[...] = s[:, :D_OUT] / denom + bg_r[...] + skip_r[...]

    return pl.pallas_call(
        body,
        grid=(N // BR,),
        in_specs=[
            pl.BlockSpec((BR, W), lambda i: (i, 0)),
            pl.BlockSpec((BR, W), lambda i: (i, 0)),
            pl.BlockSpec((BR, D_OUT), lambda i: (i, 0)),
            pl.BlockSpec((1, D_OUT), lambda i: (0, 0)),
        ],
        out_specs=pl.BlockSpec((BR, D_OUT), lambda i: (i, 0)),
        out_shape=jax.ShapeDtypeStruct((N, D_OUT), jnp.float32),
    )(q0, q1, skip, b_g)


def kernel(x, edge_index, W_sage_l, W_sage_r, b_sage, W_gat, att_src,
           att_dst, b_gat, W_lin, b_lin):
    src = edge_index[0].astype(jnp.int32)
    dst = edge_index[1].astype(jnp.int32)
    pad = E_PAD - E
    src_p = jnp.concatenate([src, jnp.zeros((pad,), jnp.int32)])
    dst_p = jnp.concatenate([dst, jnp.full((pad,), N, jnp.int32)])
    x_pad = jnp.concatenate([x, jnp.ones((N, W - D_IN), jnp.float32)], axis=1)
    zeros = jnp.zeros((N_ACC, W), jnp.float32)

    p = _sage_sc(x_pad, src_p, dst_p, zeros)
    hw_pad, skip, aux = _tc_a(
        p[0, :N], p[1, :N], x, W_sage_l, W_sage_r, b_sage.reshape(1, -1),
        W_gat, att_src.reshape(1, -1), att_dst.reshape(1, -1), W_lin,
        b_lin.reshape(1, -1))
    a_s = jnp.pad(aux[:, 0], (0, N_ACC - N))
    a_d = jnp.pad(aux[:, 1], (0, N_ACC - N))
    q = _gat_sc(hw_pad, src_p, dst_p, a_s, a_d, zeros)
    out = _tc_b(q[0, :N], q[1, :N], skip, b_gat.reshape(1, -1))
    return out
